# baseline (speedup 1.0000x reference)
import functools

import jax
import jax.numpy as jnp
from jax import lax
from jax.experimental import pallas as pl
from jax.experimental.pallas import tpu as pltpu
from jax.experimental.pallas import tpu_sc as plsc

NC = 2
NS = 16
NW = NC * NS
NSPLIT = 4


@functools.lru_cache(maxsize=None)
def _make_gather(n, vocab, d, chunk):
    per_w = n // NW
    n_chunks = per_w // chunk
    sub = chunk // NSPLIT
    assert per_w % chunk == 0 and chunk % NSPLIT == 0 and sub % 8 == 0
    mesh = plsc.VectorSubcoreMesh(
        core_axis_name="c", subcore_axis_name="s",
        num_cores=NC, num_subcores=NS,
    )

    @functools.partial(
        pl.kernel,
        out_type=jax.ShapeDtypeStruct((n, d), jnp.float32),
        mesh=mesh,
        scratch_types=(
            [pltpu.VMEM((sub,), jnp.int32) for _ in range(NSPLIT)]
            + [pltpu.VMEM((sub, d), jnp.float32) for _ in range(NSPLIT)]
            + [pltpu.SemaphoreType.DMA]
        ),
        compiler_params=pltpu.CompilerParams(use_tc_tiling_on_sc=False),
    )
    def gather(x_hbm, table_hbm, out_hbm, *scratch):
        idx_v = scratch[:NSPLIT]
        rows_v = scratch[NSPLIT:2 * NSPLIT]
        sem = scratch[2 * NSPLIT]
        wid = lax.axis_index("s") * NC + lax.axis_index("c")
        base = wid * per_w

        @pl.loop(0, n_chunks)
        def _(i):
            off = base + i * chunk
            for g in range(NSPLIT):
                pltpu.sync_copy(
                    x_hbm.at[pl.ds(off + g * sub, sub)], idx_v[g])
            descs = [
                pltpu.make_async_copy(
                    table_hbm.at[idx_v[g]], rows_v[g], sem)
                for g in range(NSPLIT)
            ]
            for d_ in descs:
                d_.start()
            for d_ in descs:
                d_.wait()
            for g in range(NSPLIT):
                pltpu.sync_copy(
                    rows_v[g], out_hbm.at[pl.ds(off + g * sub, sub)])

    return gather


@jax.jit
def kernel(x, table):
    b, t = x.shape
    vocab, d = table.shape
    xf = x.reshape(-1).astype(jnp.int32)
    out = _make_gather(b * t, vocab, d, 12800)(xf, table)
    return out.reshape(b, t, d)
